# baseline (speedup 1.0000x reference)
import jax
import jax.numpy as jnp
from jax import lax
from jax.experimental import pallas as pl
from jax.experimental.pallas import tpu as pltpu
from jax.experimental.pallas import tpu_sc as plsc

VOCAB = 1000000
DIM = 32
B, T, L = 4096, 50, 20
N = B * T
NC, NS = 2, 16
NW = NC * NS
ROWS_PER_W = N // NW
R = 800
NCHUNK = ROWS_PER_W // R
IDXW = L * R


def _body(idx_hbm, table_hbm, out_hbm,
          raw0, raw1, idx0, idx1, acc0, acc1,
          gsem0, gsem1, isem0, isem1, osem0, osem1):
    wid = lax.axis_index("s") * NC + lax.axis_index("c")
    zeros = jnp.zeros((16,), jnp.float32)
    lanes = lax.iota(jnp.int32, 16) * L

    def zero(acc):
        def zrow(r, _):
            acc[r, pl.ds(0, 16)] = zeros
            acc[r, pl.ds(16, 16)] = zeros
            return 0
        lax.fori_loop(0, R, zrow, 0)

    def transpose(raw, idx):
        def jstep(j, _):
            base = j * 16 * L
            for l in range(L):
                v = plsc.load_gather(raw, [lanes + (base + l)])
                idx[l, pl.ds(j * 16, 16)] = v
            return 0
        lax.fori_loop(0, R // 16, jstep, 0)

    def fire(idx, acc, gsem):
        return [pltpu.async_copy(table_hbm.at[idx.at[l]], acc, gsem, add=True)
                for l in range(L)]

    def drain(copies):
        for c in copies:
            c.wait()

    def idx_async(g, raw, isem):
        base = (wid * NCHUNK + g) * R
        return pltpu.async_copy(idx_hbm.at[pl.ds(base * L, IDXW)], raw, isem)

    def out_async(g, acc, osem):
        base = (wid * NCHUNK + g) * R
        return pltpu.async_copy(acc, out_hbm.at[pl.ds(base, R), :], osem)

    def wait_idx(raw, isem):
        pltpu.make_async_copy(idx_hbm.at[pl.ds(0, IDXW)], raw, isem).wait()

    def wait_out(acc, osem):
        pltpu.make_async_copy(acc, out_hbm.at[pl.ds(0, R), :], osem).wait()

    idx_async(0, raw0, isem0).wait()
    transpose(raw0, idx0)
    zero(acc0)
    c0 = fire(idx0, acc0, gsem0)
    idx_async(1, raw1, isem1)

    def step(it, carry):
        g = it * 2
        wait_idx(raw1, isem1)
        transpose(raw1, idx1)

        @pl.when(it > 0)
        def _():
            wait_out(acc1, osem1)
        zero(acc1)
        c1 = fire(idx1, acc1, gsem1)
        drain(c0)
        out_async(g, acc0, osem0)

        @pl.when(g + 2 < NCHUNK)
        def _():
            idx_async(g + 2, raw0, isem0)
            wait_idx(raw0, isem0)
            transpose(raw0, idx0)
            wait_out(acc0, osem0)
            zero(acc0)
            fire(idx0, acc0, gsem0)
        drain(c1)
        out_async(g + 1, acc1, osem1)

        @pl.when(g + 3 < NCHUNK)
        def _():
            idx_async(g + 3, raw1, isem1)
        return carry

    lax.fori_loop(0, NCHUNK // 2, step, 0)
    wait_out(acc0, osem0)
    wait_out(acc1, osem1)


@jax.jit
def _run(idx_flat, table):
    mesh = plsc.VectorSubcoreMesh(core_axis_name="c", subcore_axis_name="s")
    return pl.kernel(
        _body,
        out_type=jax.ShapeDtypeStruct((N, DIM), jnp.float32),
        mesh=mesh,
        scratch_types=[
            pltpu.VMEM((IDXW,), jnp.int32),
            pltpu.VMEM((IDXW,), jnp.int32),
            pltpu.VMEM((L, R), jnp.int32),
            pltpu.VMEM((L, R), jnp.int32),
            pltpu.VMEM((R, DIM), jnp.float32),
            pltpu.VMEM((R, DIM), jnp.float32),
            pltpu.SemaphoreType.DMA,
            pltpu.SemaphoreType.DMA,
            pltpu.SemaphoreType.DMA,
            pltpu.SemaphoreType.DMA,
            pltpu.SemaphoreType.DMA,
            pltpu.SemaphoreType.DMA,
        ],
        compiler_params=pltpu.CompilerParams(
            use_tc_tiling_on_sc=False, needs_layout_passes=False),
    )(idx_flat, table)


def kernel(sentences, table):
    out = _run(sentences.reshape(-1), table)
    return out.reshape(B, T, DIM)


# speedup vs baseline: 1.1040x; 1.1040x over previous
import jax
import jax.numpy as jnp
from jax import lax
from jax.experimental import pallas as pl
from jax.experimental.pallas import tpu as pltpu
from jax.experimental.pallas import tpu_sc as plsc

VOCAB = 1000000
DIM = 32
B, T, L = 4096, 50, 20
NC, NS = 2, 16
NW = NC * NS
BW = 16
R = BW * T
B_PER_W = B // NW
NCHUNK = B_PER_W // BW


def _body(idx_hbm, table_hbm, out_hbm,
          raw0, raw1, idx0, idx1, acc0, acc1,
          gsem0, gsem1, isem0, isem1, osem0, osem1):
    wid = lax.axis_index("s") * NC + lax.axis_index("c")
    zeros = jnp.zeros((16,), jnp.float32)
    iota = lax.iota(jnp.int32, 16)

    def zero(acc):
        def zrow(r, _):
            acc[r, pl.ds(0, 16)] = zeros
            acc[r, pl.ds(16, 16)] = zeros
            return 0
        lax.fori_loop(0, R, zrow, 0)

    def transpose(raw, idx):
        def jstep(j, _):
            r = j * 16 + iota
            d0 = r // T
            d1 = r % T
            for l in range(L):
                v = plsc.load_gather(raw, [d0, d1, jnp.full((16,), l, jnp.int32)])
                idx[l, pl.ds(j * 16, 16)] = v
            return 0
        lax.fori_loop(0, R // 16, jstep, 0)

    def fire(idx, acc, gsem):
        return [pltpu.async_copy(table_hbm.at[idx.at[l]], acc, gsem, add=True)
                for l in range(L)]

    def drain(copies):
        for c in copies:
            c.wait()

    def idx_async(g, raw, isem):
        b0 = (wid * NCHUNK + g) * BW
        return pltpu.async_copy(idx_hbm.at[pl.ds(b0, BW)], raw, isem)

    def out_async(g, acc, osem):
        b0 = (wid * NCHUNK + g) * BW
        for i in range(BW):
            pltpu.async_copy(acc.at[pl.ds(i * T, T)], out_hbm.at[b0 + i], osem)

    def wait_idx(raw, isem):
        pltpu.make_async_copy(idx_hbm.at[pl.ds(0, BW)], raw, isem).wait()

    def wait_out(acc, osem):
        for i in range(BW):
            pltpu.make_async_copy(acc.at[pl.ds(0, T)], out_hbm.at[0], osem).wait()

    idx_async(0, raw0, isem0).wait()
    transpose(raw0, idx0)
    zero(acc0)
    c0 = fire(idx0, acc0, gsem0)
    idx_async(1, raw1, isem1)

    def step(it, carry):
        g = it * 2
        wait_idx(raw1, isem1)
        transpose(raw1, idx1)

        @pl.when(it > 0)
        def _():
            wait_out(acc1, osem1)
        zero(acc1)
        c1 = fire(idx1, acc1, gsem1)
        drain(c0)
        out_async(g, acc0, osem0)

        @pl.when(g + 2 < NCHUNK)
        def _():
            idx_async(g + 2, raw0, isem0)
            wait_idx(raw0, isem0)
            transpose(raw0, idx0)
            wait_out(acc0, osem0)
            zero(acc0)
            fire(idx0, acc0, gsem0)
        drain(c1)
        out_async(g + 1, acc1, osem1)

        @pl.when(g + 3 < NCHUNK)
        def _():
            idx_async(g + 3, raw1, isem1)
        return carry

    lax.fori_loop(0, NCHUNK // 2, step, 0)
    wait_out(acc0, osem0)
    wait_out(acc1, osem1)


@jax.jit
def _run(sentences, table):
    mesh = plsc.VectorSubcoreMesh(core_axis_name="c", subcore_axis_name="s")
    return pl.kernel(
        _body,
        out_type=jax.ShapeDtypeStruct((B, T, DIM), jnp.float32),
        mesh=mesh,
        scratch_types=[
            pltpu.VMEM((BW, T, L), jnp.int32),
            pltpu.VMEM((BW, T, L), jnp.int32),
            pltpu.VMEM((L, R), jnp.int32),
            pltpu.VMEM((L, R), jnp.int32),
            pltpu.VMEM((R, DIM), jnp.float32),
            pltpu.VMEM((R, DIM), jnp.float32),
            pltpu.SemaphoreType.DMA,
            pltpu.SemaphoreType.DMA,
            pltpu.SemaphoreType.DMA,
            pltpu.SemaphoreType.DMA,
            pltpu.SemaphoreType.DMA,
            pltpu.SemaphoreType.DMA,
        ],
        compiler_params=pltpu.CompilerParams(
            use_tc_tiling_on_sc=False, needs_layout_passes=False),
    )(sentences, table)


def kernel(sentences, table):
    return _run(sentences, table)


# speedup vs baseline: 1.2688x; 1.1492x over previous
import jax
import jax.numpy as jnp
from jax import lax
from jax.experimental import pallas as pl
from jax.experimental.pallas import tpu as pltpu
from jax.experimental.pallas import tpu_sc as plsc

VOCAB = 1000000
DIM = 32
B, T, L = 4096, 50, 20
NC, NS = 2, 16
NW = NC * NS
BW = 16
R = BW * T
B_PER_W = B // NW
NCHUNK = B_PER_W // BW


def _body(idx_hbm, table_hbm, out_hbm,
          raw0, raw1, idx0, idx1, acc0, acc1,
          gsem0, gsem1, isem0, isem1, osem0, osem1):
    wid = lax.axis_index("s") * NC + lax.axis_index("c")
    zeros = jnp.zeros((16,), jnp.float32)
    iota = lax.iota(jnp.int32, 16)

    def zero(acc):
        def zrow(r, _):
            acc[r, pl.ds(0, 16)] = zeros
            acc[r, pl.ds(16, 16)] = zeros
            return 0
        lax.fori_loop(0, R, zrow, 0)

    def transpose(raw, idx):
        def jstep(j, _):
            r = j * 16 + iota
            d_i = r // T
            d_t = r % T
            for l in range(L):
                v = plsc.load_gather(raw, [jnp.full((16,), l, jnp.int32), d_t, d_i])
                idx[l, pl.ds(j * 16, 16)] = v
            return 0
        lax.fori_loop(0, R // 16, jstep, 0)

    def fire(idx, acc, gsem):
        return [pltpu.async_copy(table_hbm.at[idx.at[l]], acc, gsem, add=True)
                for l in range(L)]

    def drain(copies):
        for c in copies:
            c.wait()

    def idx_async(g, raw, isem):
        b0 = (wid * NCHUNK + g) * BW
        return pltpu.async_copy(idx_hbm.at[:, :, pl.ds(b0, BW)], raw, isem)

    def out_async(g, acc, osem):
        b0 = (wid * NCHUNK + g) * BW
        for i in range(BW):
            pltpu.async_copy(acc.at[pl.ds(i * T, T)], out_hbm.at[b0 + i], osem)

    def wait_idx(raw, isem):
        pltpu.make_async_copy(idx_hbm.at[:, :, pl.ds(0, BW)], raw, isem).wait()

    def wait_out(acc, osem):
        for i in range(BW):
            pltpu.make_async_copy(acc.at[pl.ds(0, T)], out_hbm.at[0], osem).wait()

    idx_async(0, raw0, isem0).wait()
    transpose(raw0, idx0)
    zero(acc0)
    c0 = fire(idx0, acc0, gsem0)
    idx_async(1, raw1, isem1)

    def step(it, carry):
        g = it * 2
        wait_idx(raw1, isem1)
        transpose(raw1, idx1)

        @pl.when(it > 0)
        def _():
            wait_out(acc1, osem1)
        zero(acc1)
        c1 = fire(idx1, acc1, gsem1)
        drain(c0)
        out_async(g, acc0, osem0)

        @pl.when(g + 2 < NCHUNK)
        def _():
            idx_async(g + 2, raw0, isem0)
            wait_idx(raw0, isem0)
            transpose(raw0, idx0)
            wait_out(acc0, osem0)
            zero(acc0)
            fire(idx0, acc0, gsem0)
        drain(c1)
        out_async(g + 1, acc1, osem1)

        @pl.when(g + 3 < NCHUNK)
        def _():
            idx_async(g + 3, raw1, isem1)
        return carry

    lax.fori_loop(0, NCHUNK // 2, step, 0)
    wait_out(acc0, osem0)
    wait_out(acc1, osem1)


@jax.jit
def _run(sentences, table):
    mesh = plsc.VectorSubcoreMesh(core_axis_name="c", subcore_axis_name="s")
    return pl.kernel(
        _body,
        out_type=jax.ShapeDtypeStruct((B, T, DIM), jnp.float32),
        mesh=mesh,
        scratch_types=[
            pltpu.VMEM((L, T, BW), jnp.int32),
            pltpu.VMEM((L, T, BW), jnp.int32),
            pltpu.VMEM((L, R), jnp.int32),
            pltpu.VMEM((L, R), jnp.int32),
            pltpu.VMEM((R, DIM), jnp.float32),
            pltpu.VMEM((R, DIM), jnp.float32),
            pltpu.SemaphoreType.DMA,
            pltpu.SemaphoreType.DMA,
            pltpu.SemaphoreType.DMA,
            pltpu.SemaphoreType.DMA,
            pltpu.SemaphoreType.DMA,
            pltpu.SemaphoreType.DMA,
        ],
        compiler_params=pltpu.CompilerParams(
            use_tc_tiling_on_sc=False, needs_layout_passes=False),
    )(sentences, table)


def kernel(sentences, table):
    return _run(sentences.transpose(2, 1, 0), table)
